# baseline (speedup 1.0000x reference)
import functools

import jax
import jax.numpy as jnp
from jax import lax
from jax.experimental import pallas as pl
from jax.experimental.pallas import tpu as pltpu
from jax.experimental.pallas import tpu_sc as plsc

N = 10000
E = 320000
D = 128
NC = 2
NS = 16
NW = NC * NS
C = 128
CHUNKS = 80
TE = CHUNKS * C
E_PAD = NW * TE
NACC = 10112
RPT = NACC // NS
BR = 5000

_mesh = plsc.VectorSubcoreMesh(core_axis_name="c", subcore_axis_name="s")


@functools.partial(
    pl.kernel,
    out_type=jax.ShapeDtypeStruct((NC, NACC, D), jnp.float32),
    mesh=_mesh,
    scratch_types=[
        pltpu.VMEM((CHUNKS // 2, C), jnp.int32),
        pltpu.VMEM((CHUNKS // 2, C), jnp.int32),
        pltpu.VMEM((C, D), jnp.float32),
        pltpu.VMEM((C, D), jnp.float32),
        pltpu.VMEM_SHARED((NACC, D), jnp.float32),
        pltpu.SemaphoreType.DMA,
        pltpu.SemaphoreType.DMA,
        pltpu.SemaphoreType.DMA,
        pltpu.SemaphoreType.DMA,
    ],
)
def _sc_gather_scatter(x_hbm, src_hbm, dst_hbm, init_hbm, out_hbm,
                       src_v, dst_v, rows0_v, rows1_v, acc_sh, g0, g1, s0, s1):
    c = lax.axis_index("c")
    s = lax.axis_index("s")
    wid = c * NS + s

    pltpu.sync_copy(init_hbm.at[pl.ds(s * RPT, RPT)],
                    acc_sh.at[pl.ds(s * RPT, RPT)])
    plsc.subcore_barrier()

    HC = CHUNKS // 2
    for h in range(2):
        pltpu.sync_copy(src_hbm.at[wid, pl.ds(h * HC, HC)], src_v)
        pltpu.sync_copy(dst_hbm.at[wid, pl.ds(h * HC, HC)], dst_v)

        pltpu.async_copy(x_hbm.at[src_v.at[0]], rows0_v, g0)
        pltpu.async_copy(x_hbm.at[src_v.at[1]], rows1_v, g1)

        def body(i, carry):
            j = 2 * i
            pltpu.make_async_copy(x_hbm.at[src_v.at[j]], rows0_v, g0).wait()
            pltpu.async_copy(rows0_v, acc_sh.at[dst_v.at[j]], s0, add=True)
            pltpu.make_async_copy(x_hbm.at[src_v.at[j + 1]], rows1_v, g1).wait()
            pltpu.async_copy(rows1_v, acc_sh.at[dst_v.at[j + 1]], s1, add=True)

            pltpu.make_async_copy(rows0_v, acc_sh.at[dst_v.at[j]], s0).wait()

            @pl.when(i < HC // 2 - 1)
            def _():
                pltpu.async_copy(x_hbm.at[src_v.at[j + 2]], rows0_v, g0)

            pltpu.make_async_copy(rows1_v, acc_sh.at[dst_v.at[j + 1]], s1).wait()

            @pl.when(i < HC // 2 - 1)
            def _():
                pltpu.async_copy(x_hbm.at[src_v.at[j + 3]], rows1_v, g1)

            return carry

        lax.fori_loop(0, HC // 2, body, 0)

    plsc.subcore_barrier()
    pltpu.sync_copy(acc_sh.at[pl.ds(s * RPT, RPT)],
                    out_hbm.at[c, pl.ds(s * RPT, RPT)])


def _mlp_body(x_ref, acc_ref, w1_ref, b1_ref, w2_ref, b2_ref, o_ref):
    h = x_ref[...] + acc_ref[0] + acc_ref[1]
    t = jnp.dot(h, w1_ref[...], preferred_element_type=jnp.float32) + b1_ref[...]
    t = jnp.where(t >= 0, t, 0.01 * t)
    t = jnp.dot(t, w2_ref[...], preferred_element_type=jnp.float32) + b2_ref[...]
    o_ref[...] = jnp.where(t >= 0, t, 0.01 * t)


def kernel(x, edge_index, W1, b1, W2, b2):
    src = edge_index[0]
    dst = edge_index[1]
    pad = E_PAD - E
    src_p = jnp.concatenate([src, N + jnp.arange(pad, dtype=jnp.int32) % C])
    dst_p = jnp.concatenate([dst, jnp.arange(pad, dtype=jnp.int32) % NACC])
    src_p = src_p.reshape(NW, CHUNKS, C)
    dst_p = dst_p.reshape(NW, CHUNKS, C)
    x_tbl = jnp.concatenate([x, jnp.zeros((C, D), jnp.float32)])
    init = jnp.zeros((NACC, D), jnp.float32)

    acc = _sc_gather_scatter(x_tbl, src_p, dst_p, init)

    out = pl.pallas_call(
        _mlp_body,
        grid=(N // BR,),
        in_specs=[
            pl.BlockSpec((BR, D), lambda i: (i, 0)),
            pl.BlockSpec((NC, BR, D), lambda i: (0, i, 0)),
            pl.BlockSpec((D, D), lambda i: (0, 0)),
            pl.BlockSpec((1, D), lambda i: (0, 0)),
            pl.BlockSpec((D, D), lambda i: (0, 0)),
            pl.BlockSpec((1, D), lambda i: (0, 0)),
        ],
        out_specs=pl.BlockSpec((BR, D), lambda i: (i, 0)),
        out_shape=jax.ShapeDtypeStruct((N, D), jnp.float32),
    )(x, acc, W1, b1.reshape(1, D), W2, b2.reshape(1, D))
    return out


# speedup vs baseline: 1.2359x; 1.2359x over previous
import functools

import jax
import jax.numpy as jnp
from jax import lax
from jax.experimental import pallas as pl
from jax.experimental.pallas import tpu as pltpu
from jax.experimental.pallas import tpu_sc as plsc

N = 10000
E = 320000
D = 128
NC = 2
NS = 16
NW = NC * NS
C = 128
CHUNKS = 80
TE = CHUNKS * C
E_PAD = NW * TE
NACC = 10112
RPT = NACC // NS
BR = 5000

_mesh = plsc.VectorSubcoreMesh(core_axis_name="c", subcore_axis_name="s")


@functools.partial(
    pl.kernel,
    out_type=jax.ShapeDtypeStruct((NC, NACC, D), jnp.float32),
    mesh=_mesh,
    scratch_types=[
        pltpu.VMEM((CHUNKS // 2, C), jnp.int32),
        pltpu.VMEM((CHUNKS // 2, C), jnp.int32),
        pltpu.VMEM((C, D), jnp.float32),
        pltpu.VMEM((C, D), jnp.float32),
        pltpu.VMEM_SHARED((NACC, D), jnp.float32),
        pltpu.SemaphoreType.DMA,
        pltpu.SemaphoreType.DMA,
    ],
)
def _sc_gather_scatter(x_hbm, src_hbm, dst_hbm, init_hbm, out_hbm,
                       src_v, dst_v, rows0_v, rows1_v, acc_sh, g0, g1):
    c = lax.axis_index("c")
    s = lax.axis_index("s")
    wid = c * NS + s

    pltpu.sync_copy(init_hbm.at[pl.ds(s * RPT, RPT)],
                    acc_sh.at[pl.ds(s * RPT, RPT)])
    plsc.subcore_barrier()

    HC = CHUNKS // 2
    for h in range(2):
        pltpu.sync_copy(src_hbm.at[wid, pl.ds(h * HC, HC)], src_v)
        pltpu.sync_copy(dst_hbm.at[wid, pl.ds(h * HC, HC)], dst_v)

        pltpu.async_copy(x_hbm.at[src_v.at[0]], rows0_v, g0)

        def body(i, carry):
            j = 2 * i
            pltpu.async_copy(x_hbm.at[src_v.at[j + 1]], rows1_v, g1)
            pltpu.make_async_copy(x_hbm.at[src_v.at[j]], rows0_v, g0).wait()
            pltpu.sync_copy(rows0_v, acc_sh.at[dst_v.at[j]], add=True)

            @pl.when(i < HC // 2 - 1)
            def _():
                pltpu.async_copy(x_hbm.at[src_v.at[j + 2]], rows0_v, g0)

            pltpu.make_async_copy(x_hbm.at[src_v.at[j + 1]], rows1_v, g1).wait()
            pltpu.sync_copy(rows1_v, acc_sh.at[dst_v.at[j + 1]], add=True)
            return carry

        lax.fori_loop(0, HC // 2, body, 0)

    plsc.subcore_barrier()
    pltpu.sync_copy(acc_sh.at[pl.ds(s * RPT, RPT)],
                    out_hbm.at[c, pl.ds(s * RPT, RPT)])


def _mlp_body(x_ref, acc_ref, w1_ref, b1_ref, w2_ref, b2_ref, o_ref):
    h = x_ref[...] + acc_ref[0] + acc_ref[1]
    t = jnp.dot(h, w1_ref[...], preferred_element_type=jnp.float32) + b1_ref[...]
    t = jnp.where(t >= 0, t, 0.01 * t)
    t = jnp.dot(t, w2_ref[...], preferred_element_type=jnp.float32) + b2_ref[...]
    o_ref[...] = jnp.where(t >= 0, t, 0.01 * t)


def kernel(x, edge_index, W1, b1, W2, b2):
    src = edge_index[0]
    dst = edge_index[1]
    pad = E_PAD - E
    src_p = jnp.concatenate([src, N + jnp.arange(pad, dtype=jnp.int32) % C])
    dst_p = jnp.concatenate([dst, jnp.arange(pad, dtype=jnp.int32) % NACC])
    src_p = src_p.reshape(NW, CHUNKS, C)
    dst_p = dst_p.reshape(NW, CHUNKS, C)
    x_tbl = jnp.concatenate([x, jnp.zeros((C, D), jnp.float32)])
    init = jnp.zeros((NACC, D), jnp.float32)

    acc = _sc_gather_scatter(x_tbl, src_p, dst_p, init)

    out = pl.pallas_call(
        _mlp_body,
        grid=(N // BR,),
        in_specs=[
            pl.BlockSpec((BR, D), lambda i: (i, 0)),
            pl.BlockSpec((NC, BR, D), lambda i: (0, i, 0)),
            pl.BlockSpec((D, D), lambda i: (0, 0)),
            pl.BlockSpec((1, D), lambda i: (0, 0)),
            pl.BlockSpec((D, D), lambda i: (0, 0)),
            pl.BlockSpec((1, D), lambda i: (0, 0)),
        ],
        out_specs=pl.BlockSpec((BR, D), lambda i: (i, 0)),
        out_shape=jax.ShapeDtypeStruct((N, D), jnp.float32),
    )(x, acc, W1, b1.reshape(1, D), W2, b2.reshape(1, D))
    return out


# speedup vs baseline: 1.3630x; 1.1028x over previous
import functools

import jax
import jax.numpy as jnp
from jax import lax
from jax.experimental import pallas as pl
from jax.experimental.pallas import tpu as pltpu
from jax.experimental.pallas import tpu_sc as plsc

N = 10000
E = 320000
D = 128
NC = 2
NS = 16
NW = NC * NS
C = 128
CHUNKS = 80
HC = CHUNKS // 2
EC = E // C
LAST_REAL = EC - (NW - 1) * CHUNKS
PADC = CHUNKS - LAST_REAL
NACC = 10240
RPT = NACC // NS
BR = 5000

_mesh = plsc.VectorSubcoreMesh(core_axis_name="c", subcore_axis_name="s")


@functools.partial(
    pl.kernel,
    out_type=jax.ShapeDtypeStruct((NC, NACC, D), jnp.float32),
    mesh=_mesh,
    scratch_types=[
        pltpu.VMEM((HC, C), jnp.int32),
        pltpu.VMEM((HC, C), jnp.int32),
        pltpu.VMEM((C, D), jnp.float32),
        pltpu.VMEM((C, D), jnp.float32),
        pltpu.VMEM_SHARED((NACC, D), jnp.float32),
        pltpu.SemaphoreType.DMA,
        pltpu.SemaphoreType.DMA,
    ],
)
def _sc_gather_scatter(x_hbm, eidx_hbm, pad_src_hbm, pad_dst_hbm, zeros_hbm,
                       out_hbm, src_v, dst_v, rows0_v, rows1_v, acc_sh,
                       g0, g1):
    c = lax.axis_index("c")
    s = lax.axis_index("s")
    wid = c * NS + s

    pltpu.sync_copy(zeros_hbm, acc_sh.at[pl.ds(s * RPT, RPT)])
    plsc.subcore_barrier()

    for h in range(2):
        @pl.when(wid < NW - 1)
        def _():
            base = wid * CHUNKS + h * HC
            pltpu.sync_copy(eidx_hbm.at[0, pl.ds(base, HC)], src_v)
            pltpu.sync_copy(eidx_hbm.at[1, pl.ds(base, HC)], dst_v)

        @pl.when(wid == NW - 1)
        def _():
            pltpu.sync_copy(pad_src_hbm.at[pl.ds(h * HC, HC)], src_v)
            pltpu.sync_copy(pad_dst_hbm.at[pl.ds(h * HC, HC)], dst_v)

        pltpu.async_copy(x_hbm.at[src_v.at[0]], rows0_v, g0)

        def body(i, carry):
            j = 2 * i
            pltpu.async_copy(x_hbm.at[src_v.at[j + 1]], rows1_v, g1)
            pltpu.make_async_copy(x_hbm.at[src_v.at[j]], rows0_v, g0).wait()
            pltpu.sync_copy(rows0_v, acc_sh.at[dst_v.at[j]], add=True)

            @pl.when(i < HC // 2 - 1)
            def _():
                pltpu.async_copy(x_hbm.at[src_v.at[j + 2]], rows0_v, g0)

            pltpu.make_async_copy(x_hbm.at[src_v.at[j + 1]], rows1_v, g1).wait()
            pltpu.sync_copy(rows1_v, acc_sh.at[dst_v.at[j + 1]], add=True)
            return carry

        lax.fori_loop(0, HC // 2, body, 0)

    plsc.subcore_barrier()
    pltpu.sync_copy(acc_sh.at[pl.ds(s * RPT, RPT)],
                    out_hbm.at[c, pl.ds(s * RPT, RPT)])


def _mlp_body(x_ref, acc_ref, w1_ref, b1_ref, w2_ref, b2_ref, o_ref):
    h = x_ref[...] + acc_ref[0] + acc_ref[1]
    t = jnp.dot(h, w1_ref[...], preferred_element_type=jnp.float32) + b1_ref[...]
    t = jnp.where(t >= 0, t, 0.01 * t)
    t = jnp.dot(t, w2_ref[...], preferred_element_type=jnp.float32) + b2_ref[...]
    o_ref[...] = jnp.where(t >= 0, t, 0.01 * t)


def kernel(x, edge_index, W1, b1, W2, b2):
    eidx = edge_index.reshape(2, EC, C)
    lane = jnp.arange(C, dtype=jnp.int32)
    base = (NW - 1) * CHUNKS
    pad_src = jnp.concatenate(
        [eidx[0, base:], jnp.broadcast_to(lane, (PADC, C))])
    pad_dst = jnp.concatenate(
        [eidx[1, base:], jnp.broadcast_to(N + lane, (PADC, C))])
    zeros = jnp.zeros((RPT, D), jnp.float32)

    acc = _sc_gather_scatter(x, eidx, pad_src, pad_dst, zeros)

    out = pl.pallas_call(
        _mlp_body,
        grid=(N // BR,),
        in_specs=[
            pl.BlockSpec((BR, D), lambda i: (i, 0)),
            pl.BlockSpec((NC, BR, D), lambda i: (0, i, 0)),
            pl.BlockSpec((D, D), lambda i: (0, 0)),
            pl.BlockSpec((1, D), lambda i: (0, 0)),
            pl.BlockSpec((D, D), lambda i: (0, 0)),
            pl.BlockSpec((1, D), lambda i: (0, 0)),
        ],
        out_specs=pl.BlockSpec((BR, D), lambda i: (i, 0)),
        out_shape=jax.ShapeDtypeStruct((N, D), jnp.float32),
    )(x, acc, W1, b1.reshape(1, D), W2, b2.reshape(1, D))
    return out


# speedup vs baseline: 1.3733x; 1.0076x over previous
import functools

import jax
import jax.numpy as jnp
from jax import lax
from jax.experimental import pallas as pl
from jax.experimental.pallas import tpu as pltpu
from jax.experimental.pallas import tpu_sc as plsc

N = 10000
E = 320000
D = 128
NC = 2
NS = 16
NW = NC * NS
C = 128
CHUNKS = 80
HC = CHUNKS // 2
EC = E // C
LAST_REAL = EC - (NW - 1) * CHUNKS
PADC = CHUNKS - LAST_REAL
NACC = 10240
RPT = NACC // NS
BR = 5000

_mesh = plsc.VectorSubcoreMesh(core_axis_name="c", subcore_axis_name="s")


@functools.partial(
    pl.kernel,
    out_type=jax.ShapeDtypeStruct((NC, NACC, D), jnp.float32),
    mesh=_mesh,
    scratch_types=[
        pltpu.VMEM((HC, C), jnp.int32),
        pltpu.VMEM((HC, C), jnp.int32),
        pltpu.VMEM((C, D), jnp.float32),
        pltpu.VMEM((C, D), jnp.float32),
        pltpu.VMEM_SHARED((NACC, D), jnp.float32),
        pltpu.SemaphoreType.DMA,
        pltpu.SemaphoreType.DMA,
    ],
)
def _sc_gather_scatter(x_hbm, eidx_hbm, pad_src_hbm, pad_dst_hbm, zeros_hbm,
                       out_hbm, src_v, dst_v, rows0_v, rows1_v, acc_sh,
                       g0, g1):
    c = lax.axis_index("c")
    s = lax.axis_index("s")
    wid = c * NS + s

    def stage(h):
        @pl.when(wid < NW - 1)
        def _():
            base = wid * CHUNKS + h * HC
            pltpu.sync_copy(eidx_hbm.at[0, pl.ds(base, HC)], src_v)
            pltpu.sync_copy(eidx_hbm.at[1, pl.ds(base, HC)], dst_v)

        @pl.when(wid == NW - 1)
        def _():
            pltpu.sync_copy(pad_src_hbm.at[pl.ds(h * HC, HC)], src_v)
            pltpu.sync_copy(pad_dst_hbm.at[pl.ds(h * HC, HC)], dst_v)

    stage(0)
    pltpu.async_copy(x_hbm.at[src_v.at[0]], rows0_v, g0)

    pltpu.sync_copy(zeros_hbm, acc_sh.at[pl.ds(s * RPT, RPT)])
    plsc.subcore_barrier()

    for h in range(2):
        if h == 1:
            stage(1)
            pltpu.async_copy(x_hbm.at[src_v.at[0]], rows0_v, g0)

        def body(i, carry):
            j = 2 * i
            pltpu.async_copy(x_hbm.at[src_v.at[j + 1]], rows1_v, g1)
            pltpu.make_async_copy(x_hbm.at[src_v.at[j]], rows0_v, g0).wait()
            pltpu.sync_copy(rows0_v, acc_sh.at[dst_v.at[j]], add=True)

            @pl.when(i < HC // 2 - 1)
            def _():
                pltpu.async_copy(x_hbm.at[src_v.at[j + 2]], rows0_v, g0)

            pltpu.make_async_copy(x_hbm.at[src_v.at[j + 1]], rows1_v, g1).wait()
            pltpu.sync_copy(rows1_v, acc_sh.at[dst_v.at[j + 1]], add=True)
            return carry

        lax.fori_loop(0, HC // 2, body, 0)

    plsc.subcore_barrier()
    pltpu.sync_copy(acc_sh.at[pl.ds(s * RPT, RPT)],
                    out_hbm.at[c, pl.ds(s * RPT, RPT)])


def _mlp_body(x_ref, acc_ref, w1_ref, b1_ref, w2_ref, b2_ref, o_ref):
    h = x_ref[...] + acc_ref[0] + acc_ref[1]
    t = jnp.dot(h, w1_ref[...], preferred_element_type=jnp.float32) + b1_ref[...]
    t = jnp.where(t >= 0, t, 0.01 * t)
    t = jnp.dot(t, w2_ref[...], preferred_element_type=jnp.float32) + b2_ref[...]
    o_ref[...] = jnp.where(t >= 0, t, 0.01 * t)


def kernel(x, edge_index, W1, b1, W2, b2):
    eidx = edge_index.reshape(2, EC, C)
    lane = jnp.arange(C, dtype=jnp.int32)
    base = (NW - 1) * CHUNKS
    pad_src = jnp.concatenate(
        [eidx[0, base:], jnp.broadcast_to(lane, (PADC, C))])
    pad_dst = jnp.concatenate(
        [eidx[1, base:], jnp.broadcast_to(N + lane, (PADC, C))])
    zeros = jnp.zeros((RPT, D), jnp.float32)

    acc = _sc_gather_scatter(x, eidx, pad_src, pad_dst, zeros)

    out = pl.pallas_call(
        _mlp_body,
        grid=(N // BR,),
        in_specs=[
            pl.BlockSpec((BR, D), lambda i: (i, 0)),
            pl.BlockSpec((NC, BR, D), lambda i: (0, i, 0)),
            pl.BlockSpec((D, D), lambda i: (0, 0)),
            pl.BlockSpec((1, D), lambda i: (0, 0)),
            pl.BlockSpec((D, D), lambda i: (0, 0)),
            pl.BlockSpec((1, D), lambda i: (0, 0)),
        ],
        out_specs=pl.BlockSpec((BR, D), lambda i: (i, 0)),
        out_shape=jax.ShapeDtypeStruct((N, D), jnp.float32),
    )(x, acc, W1, b1.reshape(1, D), W2, b2.reshape(1, D))
    return out
